# Initial kernel scaffold; baseline (speedup 1.0000x reference)
import jax
import jax.numpy as jnp
from jax.experimental import pallas as pl


def kernel(X, edge_index, edge_weight, params):
    raise NotImplementedError("write your pallas kernel here")


# speedup vs baseline: 15.8028x; 15.8028x over previous
import functools

import jax
import jax.numpy as jnp
from jax import lax
from jax.experimental import pallas as pl
from jax.experimental.pallas import tpu as pltpu
from jax.experimental.pallas import tpu_sc as plsc

N = 10000
NP = 10240
E = 160000
EP = 163840
D_IN = 256
D_H = 512
NC = 2
NS = 16
L = 16
CW = 128
EB = 128
RPS = NP // NS
EPS = EP // NS
EPW = EP // (NC * NS)

_mesh = plsc.VectorSubcoreMesh(core_axis_name="c", subcore_axis_name="s")
_sc_params = pltpu.CompilerParams(needs_layout_passes=False)


def _rsqrt16(x):
    i = plsc.bitcast(x, jnp.int32)
    y = plsc.bitcast(jnp.int32(0x5F3759DF) - (i >> 1), jnp.float32)
    for _ in range(3):
        y = y * (1.5 - 0.5 * x * y * y)
    return y


@functools.partial(
    pl.kernel,
    out_type=jax.ShapeDtypeStruct((EP,), jnp.float32),
    mesh=_mesh,
    compiler_params=_sc_params,
    scratch_types=[
        pltpu.VMEM((EB,), jnp.int32),
        pltpu.VMEM((EB,), jnp.float32),
        pltpu.VMEM((NP,), jnp.float32),
        pltpu.VMEM((RPS,), jnp.float32),
        pltpu.VMEM((EPW,), jnp.int32),
        pltpu.VMEM((EPW,), jnp.int32),
        pltpu.VMEM((EPW,), jnp.float32),
        pltpu.VMEM((EPW,), jnp.float32),
        pltpu.VMEM_SHARED((NP,), jnp.float32),
    ],
)
def _normw_kernel(src_hbm, dst_hbm, ew_hbm, nw_hbm,
                  didx_v, ewb_v, deg_v, zb_v,
                  s2_v, d2_v, w2_v, nw_v, degfull_sh):
    c = lax.axis_index("c")
    s = lax.axis_index("s")
    r0 = s * RPS

    def _zero(i, _):
        zb_v[pl.ds(i * L, L)] = jnp.zeros((L,), jnp.float32)
        return 0
    lax.fori_loop(0, RPS // L, _zero, 0)
    pltpu.sync_copy(zb_v, degfull_sh.at[pl.ds(r0, RPS)])
    plsc.subcore_barrier()

    def _deg(j, _):
        e0 = s * EPS + j * EB
        pltpu.sync_copy(dst_hbm.at[pl.ds(e0, EB)], didx_v)
        pltpu.sync_copy(ew_hbm.at[pl.ds(e0, EB)], ewb_v)
        pltpu.sync_copy(ewb_v, degfull_sh.at[didx_v], add=True)
        return 0
    lax.fori_loop(0, EPS // EB, _deg, 0)
    plsc.subcore_barrier()
    pltpu.sync_copy(degfull_sh, deg_v)

    wid = c * NS + s
    a0 = wid * EPW
    pltpu.sync_copy(src_hbm.at[pl.ds(a0, EPW)], s2_v)
    pltpu.sync_copy(dst_hbm.at[pl.ds(a0, EPW)], d2_v)
    pltpu.sync_copy(ew_hbm.at[pl.ds(a0, EPW)], w2_v)

    def _norm(i, _):
        sl = pl.ds(i * L, L)
        gs = plsc.load_gather(deg_v, [s2_v[sl]])
        gd = plsc.load_gather(deg_v, [d2_v[sl]])
        prod = jnp.maximum(gs, 1e-6) * jnp.maximum(gd, 1e-6)
        nw_v[sl] = w2_v[sl] * _rsqrt16(prod)
        return 0
    lax.fori_loop(0, EPW // L, _norm, 0)
    pltpu.sync_copy(nw_v, nw_hbm.at[pl.ds(a0, EPW)])


def _make_spmm(n_chunks):
    scratch = [
        pltpu.VMEM((EB,), jnp.int32),
        pltpu.VMEM((EB,), jnp.int32),
        pltpu.VMEM((EB,), jnp.float32),
        pltpu.VMEM((EB, CW), jnp.float32),
        pltpu.SemaphoreType.DMA,
        pltpu.VMEM_SHARED((NP, CW), jnp.float32),
    ]

    @functools.partial(
        pl.kernel,
        out_type=[jax.ShapeDtypeStruct((NP, CW), jnp.float32)
                  for _ in range(n_chunks)],
        mesh=_mesh,
        compiler_params=_sc_params,
        scratch_types=scratch,
    )
    def _spmm(*refs):
        xs = refs[:n_chunks]
        src_hbm, dst_hbm, nw_hbm = refs[n_chunks:n_chunks + 3]
        zout = refs[n_chunks + 3:2 * n_chunks + 3]
        sidx_v, didx_v, nwb_v, rows_v, gsem, acc_sh = refs[2 * n_chunks + 3:]
        c = lax.axis_index("c")
        s = lax.axis_index("s")
        r0 = s * RPS
        for k in range(n_chunks):
            @pl.when(c == (k % NC))
            def _():
                pltpu.sync_copy(xs[k].at[pl.ds(r0, RPS)],
                                acc_sh.at[pl.ds(r0, RPS)])
                plsc.subcore_barrier()

                def _batch(j, _):
                    e0 = s * EPS + j * EB
                    pltpu.sync_copy(src_hbm.at[pl.ds(e0, EB)], sidx_v)
                    pltpu.sync_copy(dst_hbm.at[pl.ds(e0, EB)], didx_v)
                    pltpu.sync_copy(nw_hbm.at[pl.ds(e0, EB)], nwb_v)
                    pltpu.async_copy(xs[k].at[sidx_v], rows_v, gsem).wait()

                    def _scale(e, _2):
                        w16 = plsc.load_gather(
                            nwb_v, [jnp.full((L,), e, jnp.int32)])
                        for v in range(CW // L):
                            sl = pl.ds(v * L, L)
                            rows_v[e, sl] = rows_v[e, sl] * w16
                        return 0
                    lax.fori_loop(0, EB, _scale, 0)
                    pltpu.sync_copy(rows_v, acc_sh.at[didx_v], add=True)
                    return 0
                lax.fori_loop(0, EPS // EB, _batch, 0)
                plsc.subcore_barrier()
                pltpu.sync_copy(acc_sh.at[pl.ds(r0, RPS)],
                                zout[k].at[pl.ds(r0, RPS)])
                plsc.subcore_barrier()
    return _spmm


_spmm2 = _make_spmm(D_IN // CW)
_spmm4 = _make_spmm(D_H // CW)

RB = 320


def _ln(v, g, b):
    mu = jnp.mean(v, axis=-1, keepdims=True)
    var = jnp.mean((v - mu) ** 2, axis=-1, keepdims=True)
    return (v - mu) * lax.rsqrt(var + 1e-5) * g + b


def _cell0_body(z_ref, w_ref, b_ref, wco_ref, lhg, lhb, lcg, lcb,
                h_ref, c_ref):
    g = jnp.dot(z_ref[...], w_ref[...],
                preferred_element_type=jnp.float32) + b_ref[...]
    gi = g[:, :D_H]
    gc = g[:, D_H:2 * D_H]
    go = g[:, 2 * D_H:]
    c1 = jax.nn.sigmoid(gi) * jnp.tanh(gc)
    o = jax.nn.sigmoid(go + wco_ref[...] * c1)
    h1 = o * jnp.tanh(c1)
    h_ref[...] = _ln(h1, lhg[...], lhb[...])
    c_ref[...] = _ln(c1, lcg[...], lcb[...])


def _cell1_body(z_ref, cin_ref, w_ref, b_ref, wci_ref, wcf_ref, wco_ref,
                lhg, lhb, lcg, lcb, h_ref, c_ref):
    g = jnp.dot(z_ref[...], w_ref[...],
                preferred_element_type=jnp.float32) + b_ref[...]
    cin = cin_ref[...]
    gi = g[:, :D_H]
    gf = g[:, D_H:2 * D_H]
    gc = g[:, 2 * D_H:3 * D_H]
    go = g[:, 3 * D_H:]
    i = jax.nn.sigmoid(gi + wci_ref[...] * cin)
    f = jax.nn.sigmoid(gf + wcf_ref[...] * cin)
    c2 = f * cin + i * jnp.tanh(gc)
    o = jax.nn.sigmoid(go + wco_ref[...] * c2)
    h2 = o * jnp.tanh(c2)
    h_ref[...] = _ln(h2, lhg[...], lhb[...])
    c_ref[...] = _ln(c2, lcg[...], lcb[...])


def _row_spec(w):
    return pl.BlockSpec((RB, w), lambda i: (i, 0))


def _full_spec(r, w):
    return pl.BlockSpec((r, w), lambda i: (0, 0))


_cell0_call = pl.pallas_call(
    _cell0_body,
    grid=(NP // RB,),
    in_specs=[
        _row_spec(D_IN),
        _full_spec(D_IN, 3 * D_H),
        _full_spec(1, 3 * D_H),
        _full_spec(1, D_H), _full_spec(1, D_H), _full_spec(1, D_H),
        _full_spec(1, D_H), _full_spec(1, D_H),
    ],
    out_specs=[_row_spec(D_H), _row_spec(D_H)],
    out_shape=[jax.ShapeDtypeStruct((NP, D_H), jnp.float32)] * 2,
)

_cell1_call = pl.pallas_call(
    _cell1_body,
    grid=(NP // RB,),
    in_specs=[
        _row_spec(D_H),
        _row_spec(D_H),
        _full_spec(D_H, 4 * D_H),
        _full_spec(1, 4 * D_H),
        _full_spec(1, D_H), _full_spec(1, D_H), _full_spec(1, D_H),
        _full_spec(1, D_H), _full_spec(1, D_H),
        _full_spec(1, D_H), _full_spec(1, D_H),
    ],
    out_specs=[_row_spec(D_H), _row_spec(D_H)],
    out_shape=[jax.ShapeDtypeStruct((NP, D_H), jnp.float32)] * 2,
)


def kernel(X, edge_index, edge_weight, params):
    p = params
    x = X[0]
    srcp = jnp.pad(edge_index[0], (0, EP - E))
    dstp = jnp.pad(edge_index[1], (0, EP - E), constant_values=NP - 1)
    ewp = jnp.pad(edge_weight, (0, EP - E))

    nw = _normw_kernel(srcp, dstp, ewp)

    xp = jnp.pad(x, ((0, NP - N), (0, 0)))
    xc = [xp[:, k * CW:(k + 1) * CW] for k in range(D_IN // CW)]
    z0 = jnp.concatenate(_spmm2(*xc, srcp, dstp, nw), axis=1)

    W0 = jnp.concatenate([p['Wx_i_0'], p['Wx_c_0'], p['Wx_o_0']], axis=1)
    b0 = jnp.concatenate([p['b_i_0'], p['b_c_0'], p['b_o_0']]).reshape(1, -1)
    r1 = lambda a: a.reshape(1, -1)
    h1, c1 = _cell0_call(
        z0, W0, b0, r1(p['wc_o_0']),
        r1(p['ln_h_g']), r1(p['ln_h_b']), r1(p['ln_c_g']), r1(p['ln_c_b']))

    hc = [h1[:, k * CW:(k + 1) * CW] for k in range(D_H // CW)]
    z1 = jnp.concatenate(_spmm4(*hc, srcp, dstp, nw), axis=1)

    W1 = jnp.concatenate(
        [p['Wx_%s_1' % q] + p['Wh_%s_1' % q] for q in 'ifco'], axis=1)
    b1 = jnp.concatenate([p['b_%s_1' % q] for q in 'ifco']).reshape(1, -1)
    h2, c2 = _cell1_call(
        z1, c1, W1, b1, r1(p['wc_i_1']), r1(p['wc_f_1']), r1(p['wc_o_1']),
        r1(p['ln_h_g']), r1(p['ln_h_b']), r1(p['ln_c_g']), r1(p['ln_c_b']))

    return (jnp.stack([h1[:N], h2[:N]]),
            jnp.stack([c1[:N], c2[:N]]))


# speedup vs baseline: 23.1411x; 1.4644x over previous
import functools

import jax
import jax.numpy as jnp
from jax import lax
from jax.experimental import pallas as pl
from jax.experimental.pallas import tpu as pltpu
from jax.experimental.pallas import tpu_sc as plsc

N = 10000
NP = 10240
E = 160000
EP = 163840
D_IN = 256
D_H = 512
NC = 2
NS = 16
L = 16
CW = 128
EB = 128
RPS = NP // NS
EPS = EP // NS
EPW = EP // (NC * NS)

_mesh = plsc.VectorSubcoreMesh(core_axis_name="c", subcore_axis_name="s")
_sc_params = pltpu.CompilerParams(needs_layout_passes=False)


def _rsqrt16(x):
    i = plsc.bitcast(x, jnp.int32)
    y = plsc.bitcast(jnp.int32(0x5F3759DF) - (i >> 1), jnp.float32)
    for _ in range(3):
        y = y * (1.5 - 0.5 * x * y * y)
    return y


@functools.partial(
    pl.kernel,
    out_type=jax.ShapeDtypeStruct((EP,), jnp.float32),
    mesh=_mesh,
    compiler_params=_sc_params,
    scratch_types=[
        pltpu.VMEM((EB,), jnp.int32),
        pltpu.VMEM((EB,), jnp.float32),
        pltpu.VMEM((NP,), jnp.float32),
        pltpu.VMEM((RPS,), jnp.float32),
        pltpu.VMEM((EPW,), jnp.int32),
        pltpu.VMEM((EPW,), jnp.int32),
        pltpu.VMEM((EPW,), jnp.float32),
        pltpu.VMEM((EPW,), jnp.float32),
        pltpu.VMEM_SHARED((NP,), jnp.float32),
    ],
)
def _normw_kernel(src_hbm, dst_hbm, ew_hbm, nw_hbm,
                  didx_v, ewb_v, deg_v, zb_v,
                  s2_v, d2_v, w2_v, nw_v, degfull_sh):
    c = lax.axis_index("c")
    s = lax.axis_index("s")
    r0 = s * RPS

    def _zero(i, _):
        zb_v[pl.ds(i * L, L)] = jnp.zeros((L,), jnp.float32)
        return 0
    lax.fori_loop(0, RPS // L, _zero, 0)
    pltpu.sync_copy(zb_v, degfull_sh.at[pl.ds(r0, RPS)])
    plsc.subcore_barrier()

    def _deg(j, _):
        e0 = s * EPS + j * EB
        pltpu.sync_copy(dst_hbm.at[pl.ds(e0, EB)], didx_v)
        pltpu.sync_copy(ew_hbm.at[pl.ds(e0, EB)], ewb_v)
        pltpu.sync_copy(ewb_v, degfull_sh.at[didx_v], add=True)
        return 0
    lax.fori_loop(0, EPS // EB, _deg, 0)
    plsc.subcore_barrier()
    pltpu.sync_copy(degfull_sh, deg_v)

    wid = c * NS + s
    a0 = wid * EPW
    pltpu.sync_copy(src_hbm.at[pl.ds(a0, EPW)], s2_v)
    pltpu.sync_copy(dst_hbm.at[pl.ds(a0, EPW)], d2_v)
    pltpu.sync_copy(ew_hbm.at[pl.ds(a0, EPW)], w2_v)

    def _norm(i, _):
        sl = pl.ds(i * L, L)
        gs = plsc.load_gather(deg_v, [s2_v[sl]])
        gd = plsc.load_gather(deg_v, [d2_v[sl]])
        prod = jnp.maximum(gs, 1e-6) * jnp.maximum(gd, 1e-6)
        nw_v[sl] = w2_v[sl] * _rsqrt16(prod)
        return 0
    lax.fori_loop(0, EPW // L, _norm, 0)
    pltpu.sync_copy(nw_v, nw_hbm.at[pl.ds(a0, EPW)])


def _make_spmm(n_chunks):
    NB = EPS // EB
    SLAB = 16
    NSLAB = NB // SLAB
    scratch = [
        pltpu.VMEM((SLAB, EB), jnp.int32),
        pltpu.VMEM((SLAB, EB), jnp.int32),
        pltpu.VMEM((SLAB, EB), jnp.float32),
        pltpu.VMEM((EB, CW), jnp.float32),
        pltpu.VMEM((EB, CW), jnp.float32),
        pltpu.SemaphoreType.DMA,
        pltpu.SemaphoreType.DMA,
        pltpu.VMEM_SHARED((NP, CW), jnp.float32),
    ]

    @functools.partial(
        pl.kernel,
        out_type=[jax.ShapeDtypeStruct((NP, CW), jnp.float32)
                  for _ in range(n_chunks)],
        mesh=_mesh,
        compiler_params=_sc_params,
        scratch_types=scratch,
    )
    def _spmm(*refs):
        xs = refs[:n_chunks]
        src_hbm, dst_hbm, nw_hbm = refs[n_chunks:n_chunks + 3]
        zout = refs[n_chunks + 3:2 * n_chunks + 3]
        (sidx_v, didx_v, nwb_v, rows0_v, rows1_v,
         sem0, sem1, acc_sh) = refs[2 * n_chunks + 3:]
        rows = (rows0_v, rows1_v)
        sems = (sem0, sem1)
        c = lax.axis_index("c")
        s = lax.axis_index("s")
        r0 = s * RPS

        def _scale_and_add(j, rows_v):
            def _group(g, _):
                w16 = nwb_v[j, pl.ds(g * L, L)]

                def _edge(e, _2):
                    bc = lax.gather(
                        w16, jnp.full((L, 1), e, jnp.int32),
                        lax.GatherDimensionNumbers(
                            offset_dims=(), collapsed_slice_dims=(0,),
                            start_index_map=(0,)),
                        slice_sizes=(1,),
                        mode=lax.GatherScatterMode.PROMISE_IN_BOUNDS)
                    row = g * L + e
                    for v in range(CW // L):
                        sl = pl.ds(v * L, L)
                        rows_v[row, sl] = rows_v[row, sl] * bc
                    return 0
                lax.fori_loop(0, L, _edge, 0)
                return 0
            lax.fori_loop(0, EB // L, _group, 0)
            pltpu.sync_copy(rows_v, acc_sh.at[didx_v.at[j]], add=True)

        for k in range(n_chunks):
            @pl.when(c == (k % NC))
            def _():
                pltpu.sync_copy(xs[k].at[pl.ds(r0, RPS)],
                                acc_sh.at[pl.ds(r0, RPS)])
                plsc.subcore_barrier()

                def _slab(t, _):
                    b0 = s * NB + t * SLAB
                    pltpu.sync_copy(src_hbm.at[pl.ds(b0, SLAB)], sidx_v)
                    pltpu.sync_copy(dst_hbm.at[pl.ds(b0, SLAB)], didx_v)
                    pltpu.sync_copy(nw_hbm.at[pl.ds(b0, SLAB)], nwb_v)
                    pltpu.async_copy(xs[k].at[sidx_v.at[0]], rows0_v, sem0)

                    def _pair(m, _):
                        j0 = 2 * m
                        for b in range(2):
                            j = j0 + b
                            rv, sem = rows[b], sems[b]
                            nrv, nsem = rows[1 - b], sems[1 - b]
                            pltpu.make_async_copy(
                                xs[k].at[pl.ds(0, EB)], rv, sem).wait()

                            @pl.when(j + 1 < SLAB)
                            def _():
                                pltpu.async_copy(
                                    xs[k].at[sidx_v.at[j + 1]], nrv, nsem)
                            _scale_and_add(j, rv)
                        return 0
                    lax.fori_loop(0, SLAB // 2, _pair, 0)
                    return 0
                lax.fori_loop(0, NSLAB, _slab, 0)
                plsc.subcore_barrier()
                pltpu.sync_copy(acc_sh.at[pl.ds(r0, RPS)],
                                zout[k].at[pl.ds(r0, RPS)])
                plsc.subcore_barrier()
    return _spmm


_spmm2 = _make_spmm(D_IN // CW)
_spmm4 = _make_spmm(D_H // CW)

RB = 320


def _ln(v, g, b):
    mu = jnp.mean(v, axis=-1, keepdims=True)
    var = jnp.mean((v - mu) ** 2, axis=-1, keepdims=True)
    return (v - mu) * lax.rsqrt(var + 1e-5) * g + b


def _cell0_body(z_ref, w_ref, b_ref, wco_ref, lhg, lhb, lcg, lcb,
                h_ref, c_ref):
    g = jnp.dot(z_ref[...], w_ref[...],
                preferred_element_type=jnp.float32) + b_ref[...]
    gi = g[:, :D_H]
    gc = g[:, D_H:2 * D_H]
    go = g[:, 2 * D_H:]
    c1 = jax.nn.sigmoid(gi) * jnp.tanh(gc)
    o = jax.nn.sigmoid(go + wco_ref[...] * c1)
    h1 = o * jnp.tanh(c1)
    h_ref[...] = _ln(h1, lhg[...], lhb[...])
    c_ref[...] = _ln(c1, lcg[...], lcb[...])


def _cell1_body(z_ref, cin_ref, w_ref, b_ref, wci_ref, wcf_ref, wco_ref,
                lhg, lhb, lcg, lcb, h_ref, c_ref):
    g = jnp.dot(z_ref[...], w_ref[...],
                preferred_element_type=jnp.float32) + b_ref[...]
    cin = cin_ref[...]
    gi = g[:, :D_H]
    gf = g[:, D_H:2 * D_H]
    gc = g[:, 2 * D_H:3 * D_H]
    go = g[:, 3 * D_H:]
    i = jax.nn.sigmoid(gi + wci_ref[...] * cin)
    f = jax.nn.sigmoid(gf + wcf_ref[...] * cin)
    c2 = f * cin + i * jnp.tanh(gc)
    o = jax.nn.sigmoid(go + wco_ref[...] * c2)
    h2 = o * jnp.tanh(c2)
    h_ref[...] = _ln(h2, lhg[...], lhb[...])
    c_ref[...] = _ln(c2, lcg[...], lcb[...])


def _row_spec(w):
    return pl.BlockSpec((RB, w), lambda i: (i, 0))


def _full_spec(r, w):
    return pl.BlockSpec((r, w), lambda i: (0, 0))


_cell0_call = pl.pallas_call(
    _cell0_body,
    grid=(NP // RB,),
    in_specs=[
        _row_spec(D_IN),
        _full_spec(D_IN, 3 * D_H),
        _full_spec(1, 3 * D_H),
        _full_spec(1, D_H), _full_spec(1, D_H), _full_spec(1, D_H),
        _full_spec(1, D_H), _full_spec(1, D_H),
    ],
    out_specs=[_row_spec(D_H), _row_spec(D_H)],
    out_shape=[jax.ShapeDtypeStruct((NP, D_H), jnp.float32)] * 2,
)

_cell1_call = pl.pallas_call(
    _cell1_body,
    grid=(NP // RB,),
    in_specs=[
        _row_spec(D_H),
        _row_spec(D_H),
        _full_spec(D_H, 4 * D_H),
        _full_spec(1, 4 * D_H),
        _full_spec(1, D_H), _full_spec(1, D_H), _full_spec(1, D_H),
        _full_spec(1, D_H), _full_spec(1, D_H),
        _full_spec(1, D_H), _full_spec(1, D_H),
    ],
    out_specs=[_row_spec(D_H), _row_spec(D_H)],
    out_shape=[jax.ShapeDtypeStruct((NP, D_H), jnp.float32)] * 2,
)


def kernel(X, edge_index, edge_weight, params):
    p = params
    x = X[0]
    srcp = jnp.pad(edge_index[0], (0, EP - E))
    dstp = jnp.pad(edge_index[1], (0, EP - E), constant_values=NP - 1)
    ewp = jnp.pad(edge_weight, (0, EP - E))

    nw = _normw_kernel(srcp, dstp, ewp)
    src2 = srcp.reshape(EP // EB, EB)
    dst2 = dstp.reshape(EP // EB, EB)
    nw2 = nw.reshape(EP // EB, EB)

    xp = jnp.pad(x, ((0, NP - N), (0, 0)))
    xc = [xp[:, k * CW:(k + 1) * CW] for k in range(D_IN // CW)]
    z0 = jnp.concatenate(_spmm2(*xc, src2, dst2, nw2), axis=1)

    W0 = jnp.concatenate([p['Wx_i_0'], p['Wx_c_0'], p['Wx_o_0']], axis=1)
    b0 = jnp.concatenate([p['b_i_0'], p['b_c_0'], p['b_o_0']]).reshape(1, -1)
    r1 = lambda a: a.reshape(1, -1)
    h1, c1 = _cell0_call(
        z0, W0, b0, r1(p['wc_o_0']),
        r1(p['ln_h_g']), r1(p['ln_h_b']), r1(p['ln_c_g']), r1(p['ln_c_b']))

    hc = [h1[:, k * CW:(k + 1) * CW] for k in range(D_H // CW)]
    z1 = jnp.concatenate(_spmm4(*hc, src2, dst2, nw2), axis=1)

    W1 = jnp.concatenate(
        [p['Wx_%s_1' % q] + p['Wh_%s_1' % q] for q in 'ifco'], axis=1)
    b1 = jnp.concatenate([p['b_%s_1' % q] for q in 'ifco']).reshape(1, -1)
    h2, c2 = _cell1_call(
        z1, c1, W1, b1, r1(p['wc_i_1']), r1(p['wc_f_1']), r1(p['wc_o_1']),
        r1(p['ln_h_g']), r1(p['ln_h_b']), r1(p['ln_c_g']), r1(p['ln_c_b']))

    return (jnp.stack([h1[:N], h2[:N]]),
            jnp.stack([c1[:N], c2[:N]]))


# speedup vs baseline: 23.5272x; 1.0167x over previous
import functools

import jax
import jax.numpy as jnp
from jax import lax
from jax.experimental import pallas as pl
from jax.experimental.pallas import tpu as pltpu
from jax.experimental.pallas import tpu_sc as plsc

N = 10000
NP = 10240
E = 160000
EP = 163840
D_IN = 256
D_H = 512
NC = 2
NS = 16
L = 16
CW = 128
EB = 128
EBD = 128
RPS = NP // NS
EPS = EP // NS
EPW = EP // (NC * NS)

_mesh = plsc.VectorSubcoreMesh(core_axis_name="c", subcore_axis_name="s")
_sc_params = pltpu.CompilerParams(needs_layout_passes=False)


def _rsqrt16(x):
    i = plsc.bitcast(x, jnp.int32)
    y = plsc.bitcast(jnp.int32(0x5F3759DF) - (i >> 1), jnp.float32)
    for _ in range(3):
        y = y * (1.5 - 0.5 * x * y * y)
    return y


@functools.partial(
    pl.kernel,
    out_type=jax.ShapeDtypeStruct((EP,), jnp.float32),
    mesh=_mesh,
    compiler_params=_sc_params,
    scratch_types=[
        pltpu.VMEM((16, EBD), jnp.int32),
        pltpu.VMEM((16, EBD), jnp.float32),
        pltpu.VMEM((NP,), jnp.float32),
        pltpu.VMEM((RPS,), jnp.float32),
        pltpu.VMEM((EPW,), jnp.int32),
        pltpu.VMEM((EPW,), jnp.int32),
        pltpu.VMEM((EPW,), jnp.float32),
        pltpu.VMEM((EPW,), jnp.float32),
        pltpu.SemaphoreType.DMA,
        pltpu.VMEM_SHARED((NP,), jnp.float32),
    ],
)
def _normw_kernel(src_hbm, dst_hbm, ew_hbm, dst2_hbm, ew2_hbm, nw_hbm,
                  didx_v, ewb_v, deg_v, zb_v,
                  s2_v, d2_v, w2_v, nw_v, dsem, degfull_sh):
    c = lax.axis_index("c")
    s = lax.axis_index("s")
    r0 = s * RPS

    def _zero(i, _):
        zb_v[pl.ds(i * L, L)] = jnp.zeros((L,), jnp.float32)
        return 0
    lax.fori_loop(0, RPS // L, _zero, 0)
    pltpu.sync_copy(zb_v, degfull_sh.at[pl.ds(r0, RPS)])
    plsc.subcore_barrier()

    NBD = EPS // EBD

    def _deg(t, _):
        b0 = s * NBD + t * 16
        pltpu.sync_copy(dst2_hbm.at[pl.ds(b0, 16)], didx_v)
        pltpu.sync_copy(ew2_hbm.at[pl.ds(b0, 16)], ewb_v)

        def _fire(j, _2):
            pltpu.async_copy(ewb_v.at[j], degfull_sh.at[didx_v.at[j]],
                             dsem, add=True)
            return 0
        lax.fori_loop(0, 16, _fire, 0)
        pltpu.make_async_copy(ew2_hbm.at[pl.ds(0, 16)], ewb_v, dsem).wait()
        return 0
    lax.fori_loop(0, NBD // 16, _deg, 0)
    plsc.subcore_barrier()
    pltpu.sync_copy(degfull_sh, deg_v)

    wid = c * NS + s
    a0 = wid * EPW
    pltpu.sync_copy(src_hbm.at[pl.ds(a0, EPW)], s2_v)
    pltpu.sync_copy(dst_hbm.at[pl.ds(a0, EPW)], d2_v)
    pltpu.sync_copy(ew_hbm.at[pl.ds(a0, EPW)], w2_v)

    def _norm(i, _):
        sl = pl.ds(i * L, L)
        gs = plsc.load_gather(deg_v, [s2_v[sl]])
        gd = plsc.load_gather(deg_v, [d2_v[sl]])
        prod = jnp.maximum(gs, 1e-6) * jnp.maximum(gd, 1e-6)
        nw_v[sl] = w2_v[sl] * _rsqrt16(prod)
        return 0
    lax.fori_loop(0, EPW // L, _norm, 0)
    pltpu.sync_copy(nw_v, nw_hbm.at[pl.ds(a0, EPW)])


def _make_spmm(n_chunks):
    NB = EPS // EB
    SLAB = 16
    NSLAB = NB // SLAB
    scratch = [
        pltpu.VMEM((SLAB, EB), jnp.int32),
        pltpu.VMEM((SLAB, EB), jnp.int32),
        pltpu.VMEM((SLAB, EB), jnp.float32),
        [pltpu.VMEM((EB, CW), jnp.float32) for _ in range(2)],
        [pltpu.SemaphoreType.DMA for _ in range(2)],
        [pltpu.SemaphoreType.DMA for _ in range(2)],
        pltpu.VMEM_SHARED((NP, CW), jnp.float32),
    ]

    @functools.partial(
        pl.kernel,
        out_type=[jax.ShapeDtypeStruct((NP, CW), jnp.float32)
                  for _ in range(n_chunks)],
        mesh=_mesh,
        compiler_params=_sc_params,
        scratch_types=scratch,
    )
    def _spmm(*refs):
        xs = refs[:n_chunks]
        src_hbm, dst_hbm, nw_hbm = refs[n_chunks:n_chunks + 3]
        zout = refs[n_chunks + 3:2 * n_chunks + 3]
        (sidx_v, didx_v, nwb_v, rows, gsems, ssems,
         acc_sh) = refs[2 * n_chunks + 3:]
        c = lax.axis_index("c")
        s = lax.axis_index("s")
        r0 = s * RPS

        def _scale(j, rows_v):
            def _group(g, _):
                w16 = nwb_v[j, pl.ds(g * L, L)]

                def _edge(e, _2):
                    bc = lax.gather(
                        w16, jnp.full((L, 1), e, jnp.int32),
                        lax.GatherDimensionNumbers(
                            offset_dims=(), collapsed_slice_dims=(0,),
                            start_index_map=(0,)),
                        slice_sizes=(1,),
                        mode=lax.GatherScatterMode.PROMISE_IN_BOUNDS)
                    row = g * L + e
                    for v in range(CW // L):
                        sl = pl.ds(v * L, L)
                        rows_v[row, sl] = rows_v[row, sl] * bc
                    return 0
                lax.fori_loop(0, L, _edge, 0)
                return 0
            lax.fori_loop(0, EB // L, _group, 0)

        for k in range(n_chunks):
            @pl.when(c == (k % NC))
            def _():
                pltpu.sync_copy(xs[k].at[pl.ds(r0, RPS)],
                                acc_sh.at[pl.ds(r0, RPS)])
                plsc.subcore_barrier()

                def _slab(t, _):
                    b0 = s * NB + t * SLAB
                    pltpu.sync_copy(src_hbm.at[pl.ds(b0, SLAB)], sidx_v)
                    pltpu.sync_copy(dst_hbm.at[pl.ds(b0, SLAB)], didx_v)
                    pltpu.sync_copy(nw_hbm.at[pl.ds(b0, SLAB)], nwb_v)
                    pltpu.async_copy(xs[k].at[sidx_v.at[0]], rows[0],
                                     gsems[0])

                    def _pair(m, _):
                        j0 = 2 * m
                        for b in range(2):
                            j = j0 + b
                            pltpu.make_async_copy(
                                xs[k].at[pl.ds(0, EB)], rows[b],
                                gsems[b]).wait()
                            _scale(j, rows[b])
                            pltpu.async_copy(
                                rows[b], acc_sh.at[didx_v.at[j]],
                                ssems[b], add=True)

                            @pl.when(j >= 1)
                            def _():
                                pltpu.make_async_copy(
                                    xs[k].at[pl.ds(0, EB)], rows[1 - b],
                                    ssems[1 - b]).wait()

                            @pl.when(j + 1 < SLAB)
                            def _():
                                pltpu.async_copy(
                                    xs[k].at[sidx_v.at[j + 1]], rows[1 - b],
                                    gsems[1 - b])
                        return 0
                    lax.fori_loop(0, SLAB // 2, _pair, 0)
                    pltpu.make_async_copy(
                        xs[k].at[pl.ds(0, EB)], rows[1], ssems[1]).wait()
                    return 0
                lax.fori_loop(0, NSLAB, _slab, 0)
                plsc.subcore_barrier()
                pltpu.sync_copy(acc_sh.at[pl.ds(r0, RPS)],
                                zout[k].at[pl.ds(r0, RPS)])
                plsc.subcore_barrier()
    return _spmm


_spmm2 = _make_spmm(D_IN // CW)
_spmm4 = _make_spmm(D_H // CW)

RB = 320


def _ln(v, g, b):
    mu = jnp.mean(v, axis=-1, keepdims=True)
    var = jnp.mean((v - mu) ** 2, axis=-1, keepdims=True)
    return (v - mu) * lax.rsqrt(var + 1e-5) * g + b


def _cell0_body(z_ref, w_ref, b_ref, wco_ref, lhg, lhb, lcg, lcb,
                h_ref, c_ref):
    g = jnp.dot(z_ref[...], w_ref[...],
                preferred_element_type=jnp.float32) + b_ref[...]
    gi = g[:, :D_H]
    gc = g[:, D_H:2 * D_H]
    go = g[:, 2 * D_H:]
    c1 = jax.nn.sigmoid(gi) * jnp.tanh(gc)
    o = jax.nn.sigmoid(go + wco_ref[...] * c1)
    h1 = o * jnp.tanh(c1)
    h_ref[...] = _ln(h1, lhg[...], lhb[...])
    c_ref[...] = _ln(c1, lcg[...], lcb[...])


def _cell1_body(z_ref, cin_ref, w_ref, b_ref, wci_ref, wcf_ref, wco_ref,
                lhg, lhb, lcg, lcb, h_ref, c_ref):
    g = jnp.dot(z_ref[...], w_ref[...],
                preferred_element_type=jnp.float32) + b_ref[...]
    cin = cin_ref[...]
    gi = g[:, :D_H]
    gf = g[:, D_H:2 * D_H]
    gc = g[:, 2 * D_H:3 * D_H]
    go = g[:, 3 * D_H:]
    i = jax.nn.sigmoid(gi + wci_ref[...] * cin)
    f = jax.nn.sigmoid(gf + wcf_ref[...] * cin)
    c2 = f * cin + i * jnp.tanh(gc)
    o = jax.nn.sigmoid(go + wco_ref[...] * c2)
    h2 = o * jnp.tanh(c2)
    h_ref[...] = _ln(h2, lhg[...], lhb[...])
    c_ref[...] = _ln(c2, lcg[...], lcb[...])


def _row_spec(w):
    return pl.BlockSpec((RB, w), lambda i: (i, 0))


def _full_spec(r, w):
    return pl.BlockSpec((r, w), lambda i: (0, 0))


_cell0_call = pl.pallas_call(
    _cell0_body,
    grid=(NP // RB,),
    in_specs=[
        _row_spec(D_IN),
        _full_spec(D_IN, 3 * D_H),
        _full_spec(1, 3 * D_H),
        _full_spec(1, D_H), _full_spec(1, D_H), _full_spec(1, D_H),
        _full_spec(1, D_H), _full_spec(1, D_H),
    ],
    out_specs=[_row_spec(D_H), _row_spec(D_H)],
    out_shape=[jax.ShapeDtypeStruct((NP, D_H), jnp.float32)] * 2,
)

_cell1_call = pl.pallas_call(
    _cell1_body,
    grid=(NP // RB,),
    in_specs=[
        _row_spec(D_H),
        _row_spec(D_H),
        _full_spec(D_H, 4 * D_H),
        _full_spec(1, 4 * D_H),
        _full_spec(1, D_H), _full_spec(1, D_H), _full_spec(1, D_H),
        _full_spec(1, D_H), _full_spec(1, D_H),
        _full_spec(1, D_H), _full_spec(1, D_H),
    ],
    out_specs=[_row_spec(D_H), _row_spec(D_H)],
    out_shape=[jax.ShapeDtypeStruct((NP, D_H), jnp.float32)] * 2,
)


def kernel(X, edge_index, edge_weight, params):
    p = params
    x = X[0]
    srcp = jnp.pad(edge_index[0], (0, EP - E))
    dstp = jnp.pad(edge_index[1], (0, EP - E), constant_values=NP - 1)
    ewp = jnp.pad(edge_weight, (0, EP - E))

    src2 = srcp.reshape(EP // EB, EB)
    dst2 = dstp.reshape(EP // EB, EB)
    ew2 = ewp.reshape(EP // EBD, EBD)
    nw = _normw_kernel(srcp, dstp, ewp, dstp.reshape(EP // EBD, EBD), ew2)
    nw2 = nw.reshape(EP // EB, EB)

    xp = jnp.pad(x, ((0, NP - N), (0, 0)))
    xc = [xp[:, k * CW:(k + 1) * CW] for k in range(D_IN // CW)]
    z0 = jnp.concatenate(_spmm2(*xc, src2, dst2, nw2), axis=1)

    W0 = jnp.concatenate([p['Wx_i_0'], p['Wx_c_0'], p['Wx_o_0']], axis=1)
    b0 = jnp.concatenate([p['b_i_0'], p['b_c_0'], p['b_o_0']]).reshape(1, -1)
    r1 = lambda a: a.reshape(1, -1)
    h1, c1 = _cell0_call(
        z0, W0, b0, r1(p['wc_o_0']),
        r1(p['ln_h_g']), r1(p['ln_h_b']), r1(p['ln_c_g']), r1(p['ln_c_b']))

    hc = [h1[:, k * CW:(k + 1) * CW] for k in range(D_H // CW)]
    z1 = jnp.concatenate(_spmm4(*hc, src2, dst2, nw2), axis=1)

    W1 = jnp.concatenate(
        [p['Wx_%s_1' % q] + p['Wh_%s_1' % q] for q in 'ifco'], axis=1)
    b1 = jnp.concatenate([p['b_%s_1' % q] for q in 'ifco']).reshape(1, -1)
    h2, c2 = _cell1_call(
        z1, c1, W1, b1, r1(p['wc_i_1']), r1(p['wc_f_1']), r1(p['wc_o_1']),
        r1(p['ln_h_g']), r1(p['ln_h_b']), r1(p['ln_c_g']), r1(p['ln_c_b']))

    return (jnp.stack([h1[:N], h2[:N]]),
            jnp.stack([c1[:N], c2[:N]]))


# speedup vs baseline: 25.0281x; 1.0638x over previous
import functools

import jax
import jax.numpy as jnp
from jax import lax
from jax.experimental import pallas as pl
from jax.experimental.pallas import tpu as pltpu
from jax.experimental.pallas import tpu_sc as plsc

N = 10000
NP = 10240
E = 160000
EP = 163840
D_IN = 256
D_H = 512
NC = 2
NS = 16
L = 16
CW = 128
EB = 64
EBD = 128
RPS = NP // NS
EPS = EP // NS
EPW = EP // (NC * NS)

_mesh = plsc.VectorSubcoreMesh(core_axis_name="c", subcore_axis_name="s")
_sc_params = pltpu.CompilerParams(needs_layout_passes=False)


def _rsqrt16(x):
    i = plsc.bitcast(x, jnp.int32)
    y = plsc.bitcast(jnp.int32(0x5F3759DF) - (i >> 1), jnp.float32)
    for _ in range(3):
        y = y * (1.5 - 0.5 * x * y * y)
    return y


@functools.partial(
    pl.kernel,
    out_type=jax.ShapeDtypeStruct((EP,), jnp.float32),
    mesh=_mesh,
    compiler_params=_sc_params,
    scratch_types=[
        pltpu.VMEM((16, EBD), jnp.int32),
        pltpu.VMEM((16, EBD), jnp.float32),
        pltpu.VMEM((NP,), jnp.float32),
        pltpu.VMEM((RPS,), jnp.float32),
        pltpu.VMEM((EPW,), jnp.int32),
        pltpu.VMEM((EPW,), jnp.int32),
        pltpu.VMEM((EPW,), jnp.float32),
        pltpu.VMEM((EPW,), jnp.float32),
        pltpu.SemaphoreType.DMA,
        pltpu.VMEM_SHARED((NP,), jnp.float32),
    ],
)
def _normw_kernel(src_hbm, dst_hbm, ew_hbm, dst2_hbm, ew2_hbm, nw_hbm,
                  didx_v, ewb_v, deg_v, zb_v,
                  s2_v, d2_v, w2_v, nw_v, dsem, degfull_sh):
    c = lax.axis_index("c")
    s = lax.axis_index("s")
    r0 = s * RPS

    def _zero(i, _):
        zb_v[pl.ds(i * L, L)] = jnp.zeros((L,), jnp.float32)
        return 0
    lax.fori_loop(0, RPS // L, _zero, 0)
    pltpu.sync_copy(zb_v, degfull_sh.at[pl.ds(r0, RPS)])
    plsc.subcore_barrier()

    NBD = EPS // EBD

    def _deg(t, _):
        b0 = s * NBD + t * 16
        pltpu.sync_copy(dst2_hbm.at[pl.ds(b0, 16)], didx_v)
        pltpu.sync_copy(ew2_hbm.at[pl.ds(b0, 16)], ewb_v)

        def _fire(j, _2):
            pltpu.async_copy(ewb_v.at[j], degfull_sh.at[didx_v.at[j]],
                             dsem, add=True)
            return 0
        lax.fori_loop(0, 16, _fire, 0)
        pltpu.make_async_copy(ew2_hbm.at[pl.ds(0, 16)], ewb_v, dsem).wait()
        return 0
    lax.fori_loop(0, NBD // 16, _deg, 0)
    plsc.subcore_barrier()
    pltpu.sync_copy(degfull_sh, deg_v)

    wid = c * NS + s
    a0 = wid * EPW
    pltpu.sync_copy(src_hbm.at[pl.ds(a0, EPW)], s2_v)
    pltpu.sync_copy(dst_hbm.at[pl.ds(a0, EPW)], d2_v)
    pltpu.sync_copy(ew_hbm.at[pl.ds(a0, EPW)], w2_v)

    def _norm(i, _):
        sl = pl.ds(i * L, L)
        gs = plsc.load_gather(deg_v, [s2_v[sl]])
        gd = plsc.load_gather(deg_v, [d2_v[sl]])
        prod = jnp.maximum(gs, 1e-6) * jnp.maximum(gd, 1e-6)
        nw_v[sl] = w2_v[sl] * _rsqrt16(prod)
        return 0
    lax.fori_loop(0, EPW // L, _norm, 0)
    pltpu.sync_copy(nw_v, nw_hbm.at[pl.ds(a0, EPW)])


def _make_spmm(n_chunks):
    NB = EPS // EB
    SLAB = 16
    NSLAB = NB // SLAB
    NBUF = 4
    scratch = [
        pltpu.VMEM((SLAB, EB), jnp.int32),
        pltpu.VMEM((SLAB, EB), jnp.int32),
        pltpu.VMEM((SLAB, EB), jnp.float32),
        [pltpu.VMEM((EB, CW), jnp.float32) for _ in range(NBUF)],
        [pltpu.SemaphoreType.DMA for _ in range(NBUF)],
        [pltpu.SemaphoreType.DMA for _ in range(NBUF)],
        pltpu.VMEM_SHARED((NP, CW), jnp.float32),
    ]

    @functools.partial(
        pl.kernel,
        out_type=[jax.ShapeDtypeStruct((NP, CW), jnp.float32)
                  for _ in range(n_chunks)],
        mesh=_mesh,
        compiler_params=_sc_params,
        scratch_types=scratch,
    )
    def _spmm(*refs):
        xs = refs[:n_chunks]
        src_hbm, dst_hbm, nw_hbm = refs[n_chunks:n_chunks + 3]
        zout = refs[n_chunks + 3:2 * n_chunks + 3]
        (sidx_v, didx_v, nwb_v, rows, gsems, ssems,
         acc_sh) = refs[2 * n_chunks + 3:]
        c = lax.axis_index("c")
        s = lax.axis_index("s")
        r0 = s * RPS

        def _scale(j, rows_v):
            def _group(g, _):
                w16 = nwb_v[j, pl.ds(g * L, L)]

                def _edge(e, _2):
                    bc = lax.gather(
                        w16, jnp.full((L, 1), e, jnp.int32),
                        lax.GatherDimensionNumbers(
                            offset_dims=(), collapsed_slice_dims=(0,),
                            start_index_map=(0,)),
                        slice_sizes=(1,),
                        mode=lax.GatherScatterMode.PROMISE_IN_BOUNDS)
                    row = g * L + e
                    for v in range(CW // L):
                        sl = pl.ds(v * L, L)
                        rows_v[row, sl] = rows_v[row, sl] * bc
                    return 0
                lax.fori_loop(0, L, _edge, 0)
                return 0
            lax.fori_loop(0, EB // L, _group, 0)

        for k in range(n_chunks):
            @pl.when(c == (k % NC))
            def _():
                pltpu.sync_copy(xs[k].at[pl.ds(r0, RPS)],
                                acc_sh.at[pl.ds(r0, RPS)])
                plsc.subcore_barrier()

                def _slab(t, _):
                    b0 = s * NB + t * SLAB
                    pltpu.sync_copy(src_hbm.at[pl.ds(b0, SLAB)], sidx_v)
                    pltpu.sync_copy(dst_hbm.at[pl.ds(b0, SLAB)], didx_v)
                    pltpu.sync_copy(nw_hbm.at[pl.ds(b0, SLAB)], nwb_v)
                    pltpu.async_copy(xs[k].at[sidx_v.at[0]], rows[0],
                                     gsems[0])
                    pltpu.async_copy(xs[k].at[sidx_v.at[1]], rows[1],
                                     gsems[1])

                    def _quad(m, _):
                        j0 = NBUF * m
                        for b in range(NBUF):
                            j = j0 + b
                            b2 = (b + 2) % NBUF
                            pltpu.make_async_copy(
                                xs[k].at[pl.ds(0, EB)], rows[b],
                                gsems[b]).wait()

                            @pl.when(j >= 2)
                            def _():
                                pltpu.make_async_copy(
                                    xs[k].at[pl.ds(0, EB)], rows[b2],
                                    ssems[b2]).wait()

                            @pl.when(j + 2 < SLAB)
                            def _():
                                pltpu.async_copy(
                                    xs[k].at[sidx_v.at[j + 2]], rows[b2],
                                    gsems[b2])
                            _scale(j, rows[b])
                            pltpu.async_copy(
                                rows[b], acc_sh.at[didx_v.at[j]],
                                ssems[b], add=True)
                        return 0
                    lax.fori_loop(0, SLAB // NBUF, _quad, 0)
                    pltpu.make_async_copy(
                        xs[k].at[pl.ds(0, EB)], rows[2], ssems[2]).wait()
                    pltpu.make_async_copy(
                        xs[k].at[pl.ds(0, EB)], rows[3], ssems[3]).wait()
                    return 0
                lax.fori_loop(0, NSLAB, _slab, 0)
                plsc.subcore_barrier()
                pltpu.sync_copy(acc_sh.at[pl.ds(r0, RPS)],
                                zout[k].at[pl.ds(r0, RPS)])
                plsc.subcore_barrier()
    return _spmm


_spmm2 = _make_spmm(D_IN // CW)
_spmm4 = _make_spmm(D_H // CW)

RB = 320


def _ln(v, g, b):
    mu = jnp.mean(v, axis=-1, keepdims=True)
    var = jnp.mean((v - mu) ** 2, axis=-1, keepdims=True)
    return (v - mu) * lax.rsqrt(var + 1e-5) * g + b


N0C = D_IN // CW
N1C = D_H // CW


def _cell0_body(*refs):
    zrefs = refs[:N0C]
    w_ref, b_ref, wco_ref, lhg, lhb, lcg, lcb = refs[N0C:N0C + 7]
    h_ref, c_ref = refs[N0C + 7:N0C + 9]
    hcrefs = refs[N0C + 9:]
    z = jnp.concatenate([r[...] for r in zrefs], axis=1)
    g = jnp.dot(z, w_ref[...],
                preferred_element_type=jnp.float32) + b_ref[...]
    gi = g[:, :D_H]
    gc = g[:, D_H:2 * D_H]
    go = g[:, 2 * D_H:]
    c1 = jax.nn.sigmoid(gi) * jnp.tanh(gc)
    o = jax.nn.sigmoid(go + wco_ref[...] * c1)
    h1 = o * jnp.tanh(c1)
    hln = _ln(h1, lhg[...], lhb[...])
    h_ref[...] = hln
    c_ref[...] = _ln(c1, lcg[...], lcb[...])
    for i, r in enumerate(hcrefs):
        r[...] = hln[:, i * CW:(i + 1) * CW]


def _cell1_body(*refs):
    zrefs = refs[:N1C]
    (cin_ref, w_ref, b_ref, wci_ref, wcf_ref, wco_ref,
     lhg, lhb, lcg, lcb, h_ref, c_ref) = refs[N1C:]
    z = jnp.concatenate([r[...] for r in zrefs], axis=1)
    g = jnp.dot(z, w_ref[...],
                preferred_element_type=jnp.float32) + b_ref[...]
    cin = cin_ref[...]
    gi = g[:, :D_H]
    gf = g[:, D_H:2 * D_H]
    gc = g[:, 2 * D_H:3 * D_H]
    go = g[:, 3 * D_H:]
    i = jax.nn.sigmoid(gi + wci_ref[...] * cin)
    f = jax.nn.sigmoid(gf + wcf_ref[...] * cin)
    c2 = f * cin + i * jnp.tanh(gc)
    o = jax.nn.sigmoid(go + wco_ref[...] * c2)
    h2 = o * jnp.tanh(c2)
    h_ref[...] = _ln(h2, lhg[...], lhb[...])
    c_ref[...] = _ln(c2, lcg[...], lcb[...])


def _row_spec(w):
    return pl.BlockSpec((RB, w), lambda i: (i, 0))


def _full_spec(r, w):
    return pl.BlockSpec((r, w), lambda i: (0, 0))


_cell0_call = pl.pallas_call(
    _cell0_body,
    grid=(NP // RB,),
    in_specs=[_row_spec(CW)] * N0C + [
        _full_spec(D_IN, 3 * D_H),
        _full_spec(1, 3 * D_H),
        _full_spec(1, D_H), _full_spec(1, D_H), _full_spec(1, D_H),
        _full_spec(1, D_H), _full_spec(1, D_H),
    ],
    out_specs=[_row_spec(D_H), _row_spec(D_H)] + [_row_spec(CW)] * N1C,
    out_shape=([jax.ShapeDtypeStruct((NP, D_H), jnp.float32)] * 2
               + [jax.ShapeDtypeStruct((NP, CW), jnp.float32)] * N1C),
)

_cell1_call = pl.pallas_call(
    _cell1_body,
    grid=(NP // RB,),
    in_specs=[_row_spec(CW)] * N1C + [
        _row_spec(D_H),
        _full_spec(D_H, 4 * D_H),
        _full_spec(1, 4 * D_H),
        _full_spec(1, D_H), _full_spec(1, D_H), _full_spec(1, D_H),
        _full_spec(1, D_H), _full_spec(1, D_H),
        _full_spec(1, D_H), _full_spec(1, D_H),
    ],
    out_specs=[_row_spec(D_H), _row_spec(D_H)],
    out_shape=[jax.ShapeDtypeStruct((NP, D_H), jnp.float32)] * 2,
)


def kernel(X, edge_index, edge_weight, params):
    p = params
    x = X[0]
    srcp = jnp.pad(edge_index[0], (0, EP - E))
    dstp = jnp.pad(edge_index[1], (0, EP - E), constant_values=NP - 1)
    ewp = jnp.pad(edge_weight, (0, EP - E))

    src2 = srcp.reshape(EP // EB, EB)
    dst2 = dstp.reshape(EP // EB, EB)
    ew2 = ewp.reshape(EP // EBD, EBD)
    nw = _normw_kernel(srcp, dstp, ewp, dstp.reshape(EP // EBD, EBD), ew2)
    nw2 = nw.reshape(EP // EB, EB)

    xp = jnp.pad(x, ((0, NP - N), (0, 0)))
    xc = [xp[:, k * CW:(k + 1) * CW] for k in range(D_IN // CW)]
    z0c = _spmm2(*xc, src2, dst2, nw2)

    W0 = jnp.concatenate([p['Wx_i_0'], p['Wx_c_0'], p['Wx_o_0']], axis=1)
    b0 = jnp.concatenate([p['b_i_0'], p['b_c_0'], p['b_o_0']]).reshape(1, -1)
    r1 = lambda a: a.reshape(1, -1)
    h1, c1, *hc = _cell0_call(
        *z0c, W0, b0, r1(p['wc_o_0']),
        r1(p['ln_h_g']), r1(p['ln_h_b']), r1(p['ln_c_g']), r1(p['ln_c_b']))

    z1c = _spmm4(*hc, src2, dst2, nw2)

    W1 = jnp.concatenate(
        [p['Wx_%s_1' % q] + p['Wh_%s_1' % q] for q in 'ifco'], axis=1)
    b1 = jnp.concatenate([p['b_%s_1' % q] for q in 'ifco']).reshape(1, -1)
    h2, c2 = _cell1_call(
        *z1c, c1, W1, b1, r1(p['wc_i_1']), r1(p['wc_f_1']), r1(p['wc_o_1']),
        r1(p['ln_h_g']), r1(p['ln_h_b']), r1(p['ln_c_g']), r1(p['ln_c_b']))

    return (jnp.stack([h1[:N], h2[:N]]),
            jnp.stack([c1[:N], c2[:N]]))


# speedup vs baseline: 26.5308x; 1.0600x over previous
import functools

import jax
import jax.numpy as jnp
from jax import lax
from jax.experimental import pallas as pl
from jax.experimental.pallas import tpu as pltpu
from jax.experimental.pallas import tpu_sc as plsc

N = 10000
NP = 10240
E = 160000
EP = 163840
D_IN = 256
D_H = 512
NC = 2
NS = 16
L = 16
CW = 128
EB = 64
EBD = 128
RPS = NP // NS
EPS = EP // NS
EPW = EP // (NC * NS)

_mesh = plsc.VectorSubcoreMesh(core_axis_name="c", subcore_axis_name="s")
_sc_params = pltpu.CompilerParams(needs_layout_passes=False)


def _rsqrt16(x):
    i = plsc.bitcast(x, jnp.int32)
    y = plsc.bitcast(jnp.int32(0x5F3759DF) - (i >> 1), jnp.float32)
    for _ in range(3):
        y = y * (1.5 - 0.5 * x * y * y)
    return y


@functools.partial(
    pl.kernel,
    out_type=jax.ShapeDtypeStruct((EP,), jnp.float32),
    mesh=_mesh,
    compiler_params=_sc_params,
    scratch_types=[
        pltpu.VMEM((16, EBD), jnp.int32),
        pltpu.VMEM((16, EBD), jnp.float32),
        pltpu.VMEM((NP,), jnp.float32),
        pltpu.VMEM((RPS,), jnp.float32),
        pltpu.VMEM((EPW,), jnp.int32),
        pltpu.VMEM((EPW,), jnp.int32),
        pltpu.VMEM((EPW,), jnp.float32),
        pltpu.VMEM((EPW,), jnp.float32),
        pltpu.SemaphoreType.DMA,
        pltpu.VMEM_SHARED((NP,), jnp.float32),
    ],
)
def _normw_kernel(src_hbm, dst_hbm, ew_hbm, dst2_hbm, ew2_hbm, nw_hbm,
                  didx_v, ewb_v, deg_v, zb_v,
                  s2_v, d2_v, w2_v, nw_v, dsem, degfull_sh):
    c = lax.axis_index("c")
    s = lax.axis_index("s")
    r0 = s * RPS

    def _zero(i, _):
        zb_v[pl.ds(i * L, L)] = jnp.zeros((L,), jnp.float32)
        return 0
    lax.fori_loop(0, RPS // L, _zero, 0)
    pltpu.sync_copy(zb_v, degfull_sh.at[pl.ds(r0, RPS)])
    plsc.subcore_barrier()

    NBD = EPS // EBD

    def _deg(t, _):
        b0 = s * NBD + t * 16
        pltpu.sync_copy(dst2_hbm.at[pl.ds(b0, 16)], didx_v)
        pltpu.sync_copy(ew2_hbm.at[pl.ds(b0, 16)], ewb_v)

        def _fire(j, _2):
            pltpu.async_copy(ewb_v.at[j], degfull_sh.at[didx_v.at[j]],
                             dsem, add=True)
            return 0
        lax.fori_loop(0, 16, _fire, 0)
        pltpu.make_async_copy(ew2_hbm.at[pl.ds(0, 16)], ewb_v, dsem).wait()
        return 0
    lax.fori_loop(0, NBD // 16, _deg, 0)
    plsc.subcore_barrier()
    pltpu.sync_copy(degfull_sh, deg_v)

    wid = c * NS + s
    a0 = wid * EPW
    pltpu.sync_copy(src_hbm.at[pl.ds(a0, EPW)], s2_v)
    pltpu.sync_copy(dst_hbm.at[pl.ds(a0, EPW)], d2_v)
    pltpu.sync_copy(ew_hbm.at[pl.ds(a0, EPW)], w2_v)

    def _norm(i, _):
        sl = pl.ds(i * L, L)
        gs = plsc.load_gather(deg_v, [s2_v[sl]])
        gd = plsc.load_gather(deg_v, [d2_v[sl]])
        prod = jnp.maximum(gs, 1e-6) * jnp.maximum(gd, 1e-6)
        nw_v[sl] = w2_v[sl] * _rsqrt16(prod)
        return 0
    lax.fori_loop(0, EPW // L, _norm, 0)
    pltpu.sync_copy(nw_v, nw_hbm.at[pl.ds(a0, EPW)])


def _make_spmm(n_chunks):
    NB = EPS // EB
    SLAB = 16
    NSLAB = NB // SLAB
    NBUF = 4
    scratch = [
        pltpu.VMEM((SLAB, EB), jnp.int32),
        pltpu.VMEM((SLAB, EB), jnp.int32),
        pltpu.VMEM((SLAB, EB), jnp.float32),
        [pltpu.VMEM((EB, CW), jnp.float32) for _ in range(NBUF)],
        [pltpu.SemaphoreType.DMA for _ in range(NBUF)],
        [pltpu.SemaphoreType.DMA for _ in range(NBUF)],
        pltpu.VMEM_SHARED((NP, CW), jnp.float32),
    ]

    @functools.partial(
        pl.kernel,
        out_type=[jax.ShapeDtypeStruct((NP, CW), jnp.float32)
                  for _ in range(n_chunks)],
        mesh=_mesh,
        compiler_params=_sc_params,
        scratch_types=scratch,
    )
    def _spmm(*refs):
        xs = refs[:n_chunks]
        src_hbm, dst_hbm, nw_hbm = refs[n_chunks:n_chunks + 3]
        zout = refs[n_chunks + 3:2 * n_chunks + 3]
        (sidx_v, didx_v, nwb_v, rows, gsems, ssems,
         acc_sh) = refs[2 * n_chunks + 3:]
        c = lax.axis_index("c")
        s = lax.axis_index("s")
        r0 = s * RPS

        def _scale(j, rows_v):
            def _group(g, _):
                w16 = nwb_v[j, pl.ds(g * L, L)]

                def _edge(e, _2):
                    bc = lax.gather(
                        w16, jnp.full((L, 1), e, jnp.int32),
                        lax.GatherDimensionNumbers(
                            offset_dims=(), collapsed_slice_dims=(0,),
                            start_index_map=(0,)),
                        slice_sizes=(1,),
                        mode=lax.GatherScatterMode.PROMISE_IN_BOUNDS)
                    row = g * L + e
                    for v in range(CW // L):
                        sl = pl.ds(v * L, L)
                        rows_v[row, sl] = rows_v[row, sl] * bc
                    return 0
                lax.fori_loop(0, L, _edge, 0)
                return 0
            lax.fori_loop(0, EB // L, _group, 0)

        for k in range(n_chunks):
            @pl.when(c == (k % NC))
            def _():
                pltpu.sync_copy(xs[k].at[pl.ds(r0, RPS)],
                                acc_sh.at[pl.ds(r0, RPS)])
                plsc.subcore_barrier()

                def _slab(t, _):
                    b0 = s * NB + t * SLAB
                    pltpu.sync_copy(src_hbm.at[pl.ds(b0, SLAB)], sidx_v)
                    pltpu.sync_copy(dst_hbm.at[pl.ds(b0, SLAB)], didx_v)
                    pltpu.sync_copy(nw_hbm.at[pl.ds(b0, SLAB)], nwb_v)
                    pltpu.async_copy(xs[k].at[sidx_v.at[0]], rows[0],
                                     gsems[0])
                    pltpu.async_copy(xs[k].at[sidx_v.at[1]], rows[1],
                                     gsems[1])

                    def _quad(m, _):
                        j0 = NBUF * m
                        for b in range(NBUF):
                            j = j0 + b
                            b2 = (b + 2) % NBUF
                            pltpu.make_async_copy(
                                xs[k].at[pl.ds(0, EB)], rows[b],
                                gsems[b]).wait()

                            @pl.when(j >= 2)
                            def _():
                                pltpu.make_async_copy(
                                    xs[k].at[pl.ds(0, EB)], rows[b2],
                                    ssems[b2]).wait()

                            @pl.when(j + 2 < SLAB)
                            def _():
                                pltpu.async_copy(
                                    xs[k].at[sidx_v.at[j + 2]], rows[b2],
                                    gsems[b2])
                            _scale(j, rows[b])
                            pltpu.async_copy(
                                rows[b], acc_sh.at[didx_v.at[j]],
                                ssems[b], add=True)
                        return 0
                    lax.fori_loop(0, SLAB // NBUF, _quad, 0)
                    pltpu.make_async_copy(
                        xs[k].at[pl.ds(0, EB)], rows[2], ssems[2]).wait()
                    pltpu.make_async_copy(
                        xs[k].at[pl.ds(0, EB)], rows[3], ssems[3]).wait()
                    return 0
                lax.fori_loop(0, NSLAB, _slab, 0)
                plsc.subcore_barrier()
                pltpu.sync_copy(acc_sh.at[pl.ds(r0, RPS)],
                                zout[k].at[pl.ds(r0, RPS)])
                plsc.subcore_barrier()
    return _spmm


_spmm2 = _make_spmm(D_IN // CW)
_spmm4 = _make_spmm(D_H // CW)

RB = 400


def _ln(v, g, b):
    mu = jnp.mean(v, axis=-1, keepdims=True)
    var = jnp.mean((v - mu) ** 2, axis=-1, keepdims=True)
    return (v - mu) * lax.rsqrt(var + 1e-5) * g + b


N0C = D_IN // CW
N1C = D_H // CW


def _cell0_body(*refs):
    zrefs = refs[:N0C]
    w_ref, b_ref, wco_ref, lhg, lhb, lcg, lcb = refs[N0C:N0C + 7]
    h_ref, c_ref = refs[N0C + 7:N0C + 9]
    hcrefs = refs[N0C + 9:]
    z = jnp.concatenate([r[...] for r in zrefs], axis=1)
    g = jnp.dot(z, w_ref[...],
                preferred_element_type=jnp.float32) + b_ref[...]
    gi = g[:, :D_H]
    gc = g[:, D_H:2 * D_H]
    go = g[:, 2 * D_H:]
    c1 = jax.nn.sigmoid(gi) * jnp.tanh(gc)
    o = jax.nn.sigmoid(go + wco_ref[...] * c1)
    h1 = o * jnp.tanh(c1)
    hln = _ln(h1, lhg[...], lhb[...])
    h_ref[0] = hln
    c_ref[0] = _ln(c1, lcg[...], lcb[...])
    for i, r in enumerate(hcrefs):
        r[...] = hln[:, i * CW:(i + 1) * CW]


def _cell1_body(*refs):
    zrefs = refs[:N1C]
    (hstk_in, cstk_in, w_ref, b_ref, wci_ref, wcf_ref, wco_ref,
     lhg, lhb, lcg, lcb, h_ref, c_ref) = refs[N1C:]
    del hstk_in
    z = jnp.concatenate([r[...] for r in zrefs], axis=1)
    g = jnp.dot(z, w_ref[...],
                preferred_element_type=jnp.float32) + b_ref[...]
    cin = cstk_in[0]
    gi = g[:, :D_H]
    gf = g[:, D_H:2 * D_H]
    gc = g[:, 2 * D_H:3 * D_H]
    go = g[:, 3 * D_H:]
    i = jax.nn.sigmoid(gi + wci_ref[...] * cin)
    f = jax.nn.sigmoid(gf + wcf_ref[...] * cin)
    c2 = f * cin + i * jnp.tanh(gc)
    o = jax.nn.sigmoid(go + wco_ref[...] * c2)
    h2 = o * jnp.tanh(c2)
    h_ref[0] = _ln(h2, lhg[...], lhb[...])
    c_ref[0] = _ln(c2, lcg[...], lcb[...])


def _row_spec(w):
    return pl.BlockSpec((RB, w), lambda i: (i, 0))


def _full_spec(r, w):
    return pl.BlockSpec((r, w), lambda i: (0, 0))


def _stk_spec(layer):
    return pl.BlockSpec((1, RB, D_H), lambda i, _l=layer: (_l, i, 0))


_stk_shape = jax.ShapeDtypeStruct((2, N, D_H), jnp.float32)

_cell0_call = pl.pallas_call(
    _cell0_body,
    grid=(N // RB,),
    in_specs=[_row_spec(CW)] * N0C + [
        _full_spec(D_IN, 3 * D_H),
        _full_spec(1, 3 * D_H),
        _full_spec(1, D_H), _full_spec(1, D_H), _full_spec(1, D_H),
        _full_spec(1, D_H), _full_spec(1, D_H),
    ],
    out_specs=[_stk_spec(0), _stk_spec(0)] + [_row_spec(CW)] * N1C,
    out_shape=([_stk_shape] * 2
               + [jax.ShapeDtypeStruct((NP, CW), jnp.float32)] * N1C),
)

_cell1_call = pl.pallas_call(
    _cell1_body,
    grid=(N // RB,),
    in_specs=[_row_spec(CW)] * N1C + [
        _stk_spec(0), _stk_spec(0),
        _full_spec(D_H, 4 * D_H),
        _full_spec(1, 4 * D_H),
        _full_spec(1, D_H), _full_spec(1, D_H), _full_spec(1, D_H),
        _full_spec(1, D_H), _full_spec(1, D_H),
        _full_spec(1, D_H), _full_spec(1, D_H),
    ],
    out_specs=[_stk_spec(1), _stk_spec(1)],
    out_shape=[_stk_shape] * 2,
    input_output_aliases={N1C: 0, N1C + 1: 1},
)


def kernel(X, edge_index, edge_weight, params):
    p = params
    x = X[0]
    srcp = jnp.pad(edge_index[0], (0, EP - E))
    dstp = jnp.pad(edge_index[1], (0, EP - E), constant_values=NP - 1)
    ewp = jnp.pad(edge_weight, (0, EP - E))

    src2 = srcp.reshape(EP // EB, EB)
    dst2 = dstp.reshape(EP // EB, EB)
    ew2 = ewp.reshape(EP // EBD, EBD)
    nw = _normw_kernel(srcp, dstp, ewp, dstp.reshape(EP // EBD, EBD), ew2)
    nw2 = nw.reshape(EP // EB, EB)

    xp = jnp.pad(x, ((0, NP - N), (0, 0)))
    xc = [xp[:, k * CW:(k + 1) * CW] for k in range(D_IN // CW)]
    z0c = _spmm2(*xc, src2, dst2, nw2)

    W0 = jnp.concatenate([p['Wx_i_0'], p['Wx_c_0'], p['Wx_o_0']], axis=1)
    b0 = jnp.concatenate([p['b_i_0'], p['b_c_0'], p['b_o_0']]).reshape(1, -1)
    r1 = lambda a: a.reshape(1, -1)
    hstk0, cstk0, *hc = _cell0_call(
        *z0c, W0, b0, r1(p['wc_o_0']),
        r1(p['ln_h_g']), r1(p['ln_h_b']), r1(p['ln_c_g']), r1(p['ln_c_b']))

    z1c = _spmm4(*hc, src2, dst2, nw2)

    W1 = jnp.concatenate(
        [p['Wx_%s_1' % q] + p['Wh_%s_1' % q] for q in 'ifco'], axis=1)
    b1 = jnp.concatenate([p['b_%s_1' % q] for q in 'ifco']).reshape(1, -1)
    hstk, cstk = _cell1_call(
        *z1c, hstk0, cstk0, W1, b1,
        r1(p['wc_i_1']), r1(p['wc_f_1']), r1(p['wc_o_1']),
        r1(p['ln_h_g']), r1(p['ln_h_b']), r1(p['ln_c_g']), r1(p['ln_c_b']))

    return (hstk, cstk)
